# baseline (speedup 1.0000x reference)
import functools

import jax
import jax.numpy as jnp
from jax import lax
from jax.experimental import pallas as pl
from jax.experimental.pallas import tpu as pltpu
from jax.experimental.pallas import tpu_sc as plsc

D = 128
T = 200
NC = 2
NS = 16
NW = NC * NS
L = 16
KA = 128
KB = T - KA
NBUF = 4
GD = 2


def _sc_lookup(x_flat, c_table, nb):
    per_w = nb // NW
    iters = per_w
    groups = iters // NBUF
    assert nb % NW == 0 and iters % NBUF == 0 and groups >= 2
    mesh = plsc.VectorSubcoreMesh(core_axis_name="c", subcore_axis_name="s")

    scratch = (
        [pltpu.VMEM((T * 5,), jnp.int32) for _ in range(NBUF)]
        + [pltpu.VMEM((KA,), jnp.int32) for _ in range(NBUF)]
        + [pltpu.VMEM((KB,), jnp.int32) for _ in range(NBUF)]
        + [pltpu.VMEM((T, D), jnp.float32) for _ in range(NBUF)]
        + [pltpu.SemaphoreType.DMA for _ in range(3 * NBUF)]
    )

    @functools.partial(
        pl.kernel,
        out_type=jax.ShapeDtypeStruct((nb, T, D), jnp.float32),
        mesh=mesh,
        scratch_types=scratch,
        compiler_params=pltpu.CompilerParams(needs_layout_passes=False),
    )
    def k(x_hbm, c_hbm, out_hbm, *refs):
        xv = refs[0:NBUF]
        ka = refs[NBUF:2 * NBUF]
        kb = refs[2 * NBUF:3 * NBUF]
        rows = refs[3 * NBUF:4 * NBUF]
        sx = refs[4 * NBUF:5 * NBUF]
        sg = refs[5 * NBUF:6 * NBUF]
        sw = refs[6 * NBUF:7 * NBUF]
        wid = lax.axis_index("s") * NC + lax.axis_index("c")
        wbase = wid * per_w

        def fire_xread(g, b):
            pltpu.async_copy(x_hbm.at[pl.ds((wbase + g) * (T * 5), T * 5)],
                             xv[b], sx[b])

        def wait_xread(b):
            pltpu.make_async_copy(x_hbm.at[pl.ds(0, T * 5)], xv[b], sx[b]).wait()

        def fire_gather(b):
            pltpu.async_copy(c_hbm.at[ka[b]], rows[b].at[pl.ds(0, KA), :], sg[b])
            pltpu.async_copy(c_hbm.at[kb[b]], rows[b].at[pl.ds(KA, KB), :], sg[b])

        def wait_gather(b):
            pltpu.make_async_copy(c_hbm.at[ka[b]], rows[b].at[pl.ds(0, KA), :],
                                  sg[b]).wait()
            pltpu.make_async_copy(c_hbm.at[kb[b]], rows[b].at[pl.ds(KA, KB), :],
                                  sg[b]).wait()

        def fire_write(g, b):
            pltpu.async_copy(rows[b], out_hbm.at[wbase + g], sw[b])

        def wait_write(b):
            pltpu.make_async_copy(rows[b], out_hbm.at[0], sw[b]).wait()

        lane = lax.iota(jnp.int32, L)

        def one_key(b, pos):
            kk = plsc.load_gather(xv[b], [pos * 5])
            for j in range(1, 5):
                kk = kk * 7 + plsc.load_gather(xv[b], [pos * 5 + j])
            return kk

        def compute_keys(b):
            for i in range(T // L):
                kk = one_key(b, lane + (i * L))
                if i < KA // L:
                    ka[b][pl.ds(i * L, L)] = kk
                else:
                    kb[b][pl.ds(i * L - KA, L)] = kk
            tail0 = (T // L) * L
            kk = one_key(b, jnp.minimum(lane + tail0, T - 1))
            idx = jnp.minimum(lane + (tail0 - KA), KB - 1)
            plsc.store_scatter(kb[b], [idx], kk, mask=lane < (T - tail0))

        def step(g, b, fire_read, wait_w, drain):
            wait_xread(b)
            compute_keys(b)
            if fire_read:
                fire_xread(g + NBUF, b)
            if wait_w:
                wait_write(b)
            fire_gather(b)
            if drain:
                pb = (b - GD) % NBUF
                wait_gather(pb)
                fire_write(g - GD, pb)

        for b in range(NBUF):
            fire_xread(b, b)
        for b in range(NBUF):
            step(b, b, fire_read=True, wait_w=False, drain=(b >= GD))

        def body(grp, c):
            g0 = grp * NBUF
            for b in range(NBUF):
                step(g0 + b, b, fire_read=True, wait_w=True, drain=True)
            return c

        lax.fori_loop(1, groups - 1, body, 0)

        gl = (groups - 1) * NBUF
        for b in range(NBUF):
            step(gl + b, b, fire_read=False, wait_w=True, drain=True)

        for i in range(GD):
            b = (NBUF - GD + i) % NBUF
            wait_gather(b)
            fire_write(iters - GD + i, b)
        for b in range(NBUF):
            wait_write(b)

    return k(x_flat, c_table)


def _build_combined(mt, dt, wt, ht, mnt):
    def body(m_ref, d_ref, w_ref, h_ref, mi_ref, out_ref):
        d_, w_, h_, mi_ = d_ref[...], w_ref[...], h_ref[...], mi_ref[...]
        m_row = m_ref[pl.ds(pl.program_id(0), 1), :]
        dw = (d_[:, None, :] + w_[None, :, :]).reshape(49, D)
        hm = (h_[:, None, :] + mi_[None, :, :]).reshape(49, D)
        out_ref[...] = ((m_row[0] + dw)[None, :, None, :]
                        + hm[None, None, :, :])

    row7 = pl.BlockSpec((7, D), lambda m: (0, 0))
    c4 = pl.pallas_call(
        body,
        grid=(7,),
        in_specs=[row7, row7, row7, row7, row7],
        out_specs=pl.BlockSpec((1, 49, 49, D), lambda m: (m, 0, 0, 0)),
        out_shape=jax.ShapeDtypeStruct((7, 49, 49, D), jnp.float32),
    )(mt, dt, wt, ht, mnt)
    return c4.reshape(7 ** 5, D)


def kernel(x, minute_table, hour_table, weekday_table, day_table, month_table):
    b, t, _ = x.shape
    c = _build_combined(month_table[:7], day_table[:7], weekday_table[:7],
                        hour_table[:7], minute_table[:7])
    x_flat = x.reshape(b * t * 5).astype(jnp.int32)
    return _sc_lookup(x_flat, c, b)


# speedup vs baseline: 1.0651x; 1.0651x over previous
import functools

import jax
import jax.numpy as jnp
from jax import lax
from jax.experimental import pallas as pl
from jax.experimental.pallas import tpu as pltpu
from jax.experimental.pallas import tpu_sc as plsc

D = 128
T = 200
NC = 2
NS = 16
NW = NC * NS
L = 16
KA = 128
KB = T - KA
NBUF = 2
GD = 1


def _sc_lookup(x_flat, c_table, nb):
    per_w = nb // NW
    iters = per_w
    groups = iters // NBUF
    assert nb % NW == 0 and iters % NBUF == 0 and groups >= 2
    mesh = plsc.VectorSubcoreMesh(core_axis_name="c", subcore_axis_name="s")

    scratch = (
        [pltpu.VMEM((T, 5), jnp.int32) for _ in range(NBUF)]
        + [pltpu.VMEM((KA,), jnp.int32) for _ in range(NBUF)]
        + [pltpu.VMEM((KB,), jnp.int32) for _ in range(NBUF)]
        + [pltpu.VMEM((T, D), jnp.float32) for _ in range(NBUF)]
        + [pltpu.SemaphoreType.DMA for _ in range(3 * NBUF)]
    )

    @functools.partial(
        pl.kernel,
        out_type=jax.ShapeDtypeStruct((nb, T, D), jnp.float32),
        mesh=mesh,
        scratch_types=scratch,
        compiler_params=pltpu.CompilerParams(needs_layout_passes=False),
    )
    def k(x_hbm, c_hbm, out_hbm, *refs):
        xv = refs[0:NBUF]
        ka = refs[NBUF:2 * NBUF]
        kb = refs[2 * NBUF:3 * NBUF]
        rows = refs[3 * NBUF:4 * NBUF]
        sx = refs[4 * NBUF:5 * NBUF]
        sg = refs[5 * NBUF:6 * NBUF]
        sw = refs[6 * NBUF:7 * NBUF]
        wid = lax.axis_index("s") * NC + lax.axis_index("c")
        wbase = wid * per_w

        def fire_xread(g, b):
            pltpu.async_copy(x_hbm.at[wbase + g], xv[b], sx[b])

        def wait_xread(b):
            pltpu.make_async_copy(x_hbm.at[0], xv[b], sx[b]).wait()

        def fire_gather(b):
            pltpu.async_copy(c_hbm.at[ka[b]], rows[b].at[pl.ds(0, KA), :], sg[b])
            pltpu.async_copy(c_hbm.at[kb[b]], rows[b].at[pl.ds(KA, KB), :], sg[b])

        def wait_gather(b):
            pltpu.make_async_copy(c_hbm.at[ka[b]], rows[b].at[pl.ds(0, KA), :],
                                  sg[b]).wait()
            pltpu.make_async_copy(c_hbm.at[kb[b]], rows[b].at[pl.ds(KA, KB), :],
                                  sg[b]).wait()

        def fire_write(g, b):
            pltpu.async_copy(rows[b], out_hbm.at[wbase + g], sw[b])

        def wait_write(b):
            pltpu.make_async_copy(rows[b], out_hbm.at[0], sw[b]).wait()

        lane = lax.iota(jnp.int32, L)

        def one_key(b, pos):
            kk = plsc.load_gather(xv[b], [pos, jnp.zeros((L,), jnp.int32)])
            for j in range(1, 5):
                kk = kk * 7 + plsc.load_gather(
                    xv[b], [pos, jnp.full((L,), j, jnp.int32)])
            return kk

        def compute_keys(b):
            for i in range(T // L):
                kk = one_key(b, lane + (i * L))
                if i < KA // L:
                    ka[b][pl.ds(i * L, L)] = kk
                else:
                    kb[b][pl.ds(i * L - KA, L)] = kk
            tail0 = (T // L) * L
            kk = one_key(b, jnp.minimum(lane + tail0, jnp.int32(T - 1)))
            idx = jnp.minimum(lane + (tail0 - KA), KB - 1)
            plsc.store_scatter(kb[b], [idx], kk, mask=lane < (T - tail0))

        def step(g, b, fire_read, wait_w, drain):
            wait_xread(b)
            compute_keys(b)
            if fire_read:
                fire_xread(g + NBUF, b)
            if wait_w:
                wait_write(b)
            fire_gather(b)
            if drain:
                pb = (b - GD) % NBUF
                wait_gather(pb)
                fire_write(g - GD, pb)

        for b in range(NBUF):
            fire_xread(b, b)
        for b in range(NBUF):
            step(b, b, fire_read=True, wait_w=False, drain=(b >= GD))

        def body(grp, c):
            g0 = grp * NBUF
            for b in range(NBUF):
                step(g0 + b, b, fire_read=True, wait_w=True, drain=True)
            return c

        lax.fori_loop(1, groups - 1, body, 0)

        gl = (groups - 1) * NBUF
        for b in range(NBUF):
            step(gl + b, b, fire_read=False, wait_w=True, drain=True)

        for i in range(GD):
            b = (NBUF - GD + i) % NBUF
            wait_gather(b)
            fire_write(iters - GD + i, b)
        for b in range(NBUF):
            wait_write(b)

    return k(x_flat, c_table)


def _build_combined(mt, dt, wt, ht, mnt):
    def body(m_ref, d_ref, w_ref, h_ref, mi_ref, out_ref):
        d_, w_, h_, mi_ = d_ref[...], w_ref[...], h_ref[...], mi_ref[...]
        m_row = m_ref[pl.ds(pl.program_id(0), 1), :]
        dw = (d_[:, None, :] + w_[None, :, :]).reshape(49, D)
        hm = (h_[:, None, :] + mi_[None, :, :]).reshape(49, D)
        out_ref[...] = ((m_row[0] + dw)[None, :, None, :]
                        + hm[None, None, :, :])

    row7 = pl.BlockSpec((7, D), lambda m: (0, 0))
    c4 = pl.pallas_call(
        body,
        grid=(7,),
        in_specs=[row7, row7, row7, row7, row7],
        out_specs=pl.BlockSpec((1, 49, 49, D), lambda m: (m, 0, 0, 0)),
        out_shape=jax.ShapeDtypeStruct((7, 49, 49, D), jnp.float32),
    )(mt, dt, wt, ht, mnt)
    return c4.reshape(7 ** 5, D)


def kernel(x, minute_table, hour_table, weekday_table, day_table, month_table):
    b, t, _ = x.shape
    c = _build_combined(month_table[:7], day_table[:7], weekday_table[:7],
                        hour_table[:7], minute_table[:7])
    return _sc_lookup(x.astype(jnp.int32), c, b)


# speedup vs baseline: 1.0706x; 1.0051x over previous
import functools

import jax
import jax.numpy as jnp
from jax import lax
from jax.experimental import pallas as pl
from jax.experimental.pallas import tpu as pltpu
from jax.experimental.pallas import tpu_sc as plsc

D = 128
T = 200
CPA = 96
CPB = 104
NC = 2
NS = 16
NW = NC * NS
L = 16
NBUF = 4
GD = 2


def _sc_lookup(x3d, c_table, nb):
    rows_per_w = nb // NW
    iters = rows_per_w * 2
    groups = iters // NBUF
    assert nb % NW == 0 and iters % NBUF == 0 and groups >= 2
    mesh = plsc.VectorSubcoreMesh(core_axis_name="c", subcore_axis_name="s")

    def _cp(b):
        return CPA if b % 2 == 0 else CPB

    scratch = (
        [pltpu.VMEM((_cp(b), 5), jnp.int32) for b in range(NBUF)]
        + [pltpu.VMEM((_cp(b),), jnp.int32) for b in range(NBUF)]
        + [pltpu.VMEM((_cp(b), D), jnp.float32) for b in range(NBUF)]
        + [pltpu.SemaphoreType.DMA for _ in range(3 * NBUF)]
    )

    @functools.partial(
        pl.kernel,
        out_type=jax.ShapeDtypeStruct((nb, T, D), jnp.float32),
        mesh=mesh,
        scratch_types=scratch,
        compiler_params=pltpu.CompilerParams(needs_layout_passes=False),
    )
    def k(x_hbm, c_hbm, out_hbm, *refs):
        xv = refs[0:NBUF]
        keys = refs[NBUF:2 * NBUF]
        rows = refs[2 * NBUF:3 * NBUF]
        sx = refs[3 * NBUF:4 * NBUF]
        sg = refs[4 * NBUF:5 * NBUF]
        sw = refs[5 * NBUF:6 * NBUF]
        wid = lax.axis_index("s") * NC + lax.axis_index("c")
        wrow = wid * rows_per_w

        def rh(g, b):
            return wrow + g // 2, 0 if b % 2 == 0 else CPA

        def fire_xread(g, b):
            r, h = rh(g, b)
            pltpu.async_copy(x_hbm.at[r, pl.ds(h, _cp(b)), :], xv[b], sx[b])

        def wait_xread(b):
            pltpu.make_async_copy(x_hbm.at[0, pl.ds(0, _cp(b)), :], xv[b],
                                  sx[b]).wait()

        def fire_gather(b):
            pltpu.async_copy(c_hbm.at[keys[b]], rows[b], sg[b])

        def wait_gather(b):
            pltpu.make_async_copy(c_hbm.at[keys[b]], rows[b], sg[b]).wait()

        def fire_write(g, b):
            r, h = rh(g, b)
            pltpu.async_copy(rows[b], out_hbm.at[r, pl.ds(h, _cp(b)), :], sw[b])

        def wait_write(b):
            pltpu.make_async_copy(rows[b], out_hbm.at[0, pl.ds(0, _cp(b)), :],
                                  sw[b]).wait()

        lane = lax.iota(jnp.int32, L)

        def one_key(b, pos):
            kk = plsc.load_gather(xv[b], [pos, jnp.zeros((L,), jnp.int32)])
            for j in range(1, 5):
                kk = kk * 7 + plsc.load_gather(
                    xv[b], [pos, jnp.full((L,), j, jnp.int32)])
            return kk

        def compute_keys(b):
            cp = _cp(b)
            for i in range(cp // L):
                keys[b][pl.ds(i * L, L)] = one_key(b, lane + (i * L))
            tail = cp - (cp // L) * L
            if tail:
                tail0 = (cp // L) * L
                idx = jnp.minimum(lane + tail0, jnp.int32(cp - 1))
                kk = one_key(b, idx)
                plsc.store_scatter(keys[b], [idx], kk, mask=lane < tail)

        def step(g, b, fire_read, wait_w, drain):
            wait_xread(b)
            compute_keys(b)
            if fire_read:
                fire_xread(g + NBUF, b)
            if wait_w:
                wait_write(b)
            fire_gather(b)
            if drain:
                pb = (b - GD) % NBUF
                wait_gather(pb)
                fire_write(g - GD, pb)

        for b in range(NBUF):
            fire_xread(b, b)
        for b in range(NBUF):
            step(b, b, fire_read=True, wait_w=False, drain=(b >= GD))

        def body(grp, c):
            g0 = grp * NBUF
            for b in range(NBUF):
                step(g0 + b, b, fire_read=True, wait_w=True, drain=True)
            return c

        lax.fori_loop(1, groups - 1, body, 0)

        gl = (groups - 1) * NBUF
        for b in range(NBUF):
            step(gl + b, b, fire_read=False, wait_w=True, drain=True)

        for i in range(GD):
            b = (NBUF - GD + i) % NBUF
            wait_gather(b)
            fire_write(iters - GD + i, b)
        for b in range(NBUF):
            wait_write(b)

    return k(x3d, c_table)


def _build_combined(mt, dt, wt, ht, mnt):
    def body(m_ref, d_ref, w_ref, h_ref, mi_ref, out_ref):
        d_, w_, h_, mi_ = d_ref[...], w_ref[...], h_ref[...], mi_ref[...]
        m_row = m_ref[pl.ds(pl.program_id(0), 1), :]
        dw = (d_[:, None, :] + w_[None, :, :]).reshape(49, D)
        hm = (h_[:, None, :] + mi_[None, :, :]).reshape(49, D)
        out_ref[...] = ((m_row[0] + dw)[None, :, None, :]
                        + hm[None, None, :, :])

    row7 = pl.BlockSpec((7, D), lambda m: (0, 0))
    c4 = pl.pallas_call(
        body,
        grid=(7,),
        in_specs=[row7, row7, row7, row7, row7],
        out_specs=pl.BlockSpec((1, 49, 49, D), lambda m: (m, 0, 0, 0)),
        out_shape=jax.ShapeDtypeStruct((7, 49, 49, D), jnp.float32),
    )(mt, dt, wt, ht, mnt)
    return c4.reshape(7 ** 5, D)


def kernel(x, minute_table, hour_table, weekday_table, day_table, month_table):
    b, t, _ = x.shape
    c = _build_combined(month_table[:7], day_table[:7], weekday_table[:7],
                        hour_table[:7], minute_table[:7])
    return _sc_lookup(x.astype(jnp.int32), c, b)


# speedup vs baseline: 1.0725x; 1.0018x over previous
import functools

import jax
import jax.numpy as jnp
from jax import lax
from jax.experimental import pallas as pl
from jax.experimental.pallas import tpu as pltpu
from jax.experimental.pallas import tpu_sc as plsc

D = 128
T = 200
CPA = 96
CPB = 104
NC = 2
NS = 16
NW = NC * NS
L = 16
NBUF = 4
GD = 2


def _sc_lookup(x3d, c_table, nb):
    rows_per_w = nb // NW
    iters = rows_per_w * 2
    groups = iters // NBUF
    assert nb % NW == 0 and iters % NBUF == 0 and groups >= 2
    mesh = plsc.VectorSubcoreMesh(core_axis_name="c", subcore_axis_name="s")

    def _cp(b):
        return CPA if b % 2 == 0 else CPB

    scratch = (
        [pltpu.VMEM((_cp(b), 5), jnp.int32) for b in range(NBUF)]
        + [pltpu.VMEM((_cp(b),), jnp.int32) for b in range(NBUF)]
        + [pltpu.VMEM((_cp(b), D), jnp.float32) for b in range(NBUF)]
        + [pltpu.SemaphoreType.DMA for _ in range(3 * NBUF)]
    )

    @functools.partial(
        pl.kernel,
        out_type=jax.ShapeDtypeStruct((nb, T, D), jnp.float32),
        mesh=mesh,
        scratch_types=scratch,
        compiler_params=pltpu.CompilerParams(needs_layout_passes=False,
                                             use_tc_tiling_on_sc=True),
    )
    def k(x_hbm, c_hbm, out_hbm, *refs):
        xv = refs[0:NBUF]
        keys = refs[NBUF:2 * NBUF]
        rows = refs[2 * NBUF:3 * NBUF]
        sx = refs[3 * NBUF:4 * NBUF]
        sg = refs[4 * NBUF:5 * NBUF]
        sw = refs[5 * NBUF:6 * NBUF]
        wid = lax.axis_index("s") * NC + lax.axis_index("c")
        wrow = wid * rows_per_w

        def rh(g, b):
            return wrow + g // 2, 0 if b % 2 == 0 else CPA

        def fire_xread(g, b):
            r, h = rh(g, b)
            pltpu.async_copy(x_hbm.at[r, pl.ds(h, _cp(b)), :], xv[b], sx[b])

        def wait_xread(b):
            pltpu.make_async_copy(x_hbm.at[0, pl.ds(0, _cp(b)), :], xv[b],
                                  sx[b]).wait()

        def fire_gather(b):
            pltpu.async_copy(c_hbm.at[keys[b]], rows[b], sg[b])

        def wait_gather(b):
            pltpu.make_async_copy(c_hbm.at[keys[b]], rows[b], sg[b]).wait()

        def fire_write(g, b):
            r, h = rh(g, b)
            pltpu.async_copy(rows[b], out_hbm.at[r, pl.ds(h, _cp(b)), :], sw[b])

        def wait_write(b):
            pltpu.make_async_copy(rows[b], out_hbm.at[0, pl.ds(0, _cp(b)), :],
                                  sw[b]).wait()

        lane = lax.iota(jnp.int32, L)

        def one_key(b, pos):
            kk = plsc.load_gather(xv[b], [pos, jnp.zeros((L,), jnp.int32)])
            for j in range(1, 5):
                kk = kk * 7 + plsc.load_gather(
                    xv[b], [pos, jnp.full((L,), j, jnp.int32)])
            return kk

        def compute_keys(b):
            cp = _cp(b)
            for i in range(cp // L):
                keys[b][pl.ds(i * L, L)] = one_key(b, lane + (i * L))
            tail = cp - (cp // L) * L
            if tail:
                tail0 = (cp // L) * L
                idx = jnp.minimum(lane + tail0, jnp.int32(cp - 1))
                kk = one_key(b, idx)
                plsc.store_scatter(keys[b], [idx], kk, mask=lane < tail)

        def step(g, b, fire_read, wait_w, drain):
            wait_xread(b)
            compute_keys(b)
            if fire_read:
                fire_xread(g + NBUF, b)
            if wait_w:
                wait_write(b)
            fire_gather(b)
            if drain:
                pb = (b - GD) % NBUF
                wait_gather(pb)
                fire_write(g - GD, pb)

        for b in range(NBUF):
            fire_xread(b, b)
        for b in range(NBUF):
            step(b, b, fire_read=True, wait_w=False, drain=(b >= GD))

        def body(grp, c):
            g0 = grp * NBUF
            for b in range(NBUF):
                step(g0 + b, b, fire_read=True, wait_w=True, drain=True)
            return c

        lax.fori_loop(1, groups - 1, body, 0)

        gl = (groups - 1) * NBUF
        for b in range(NBUF):
            step(gl + b, b, fire_read=False, wait_w=True, drain=True)

        for i in range(GD):
            b = (NBUF - GD + i) % NBUF
            wait_gather(b)
            fire_write(iters - GD + i, b)
        for b in range(NBUF):
            wait_write(b)

    return k(x3d, c_table)


def _build_combined(mt, dt, wt, ht, mnt):
    def body(m_ref, d_ref, w_ref, h_ref, mi_ref, out_ref):
        d_, w_, h_, mi_ = d_ref[...], w_ref[...], h_ref[...], mi_ref[...]
        m_row = m_ref[pl.ds(pl.program_id(0), 1), :]
        dw = (d_[:, None, :] + w_[None, :, :]).reshape(49, D)
        hm = (h_[:, None, :] + mi_[None, :, :]).reshape(49, D)
        out_ref[...] = ((m_row[0] + dw)[None, :, None, :]
                        + hm[None, None, :, :])

    row7 = pl.BlockSpec((7, D), lambda m: (0, 0))
    c4 = pl.pallas_call(
        body,
        grid=(7,),
        in_specs=[row7, row7, row7, row7, row7],
        out_specs=pl.BlockSpec((1, 49, 49, D), lambda m: (m, 0, 0, 0)),
        out_shape=jax.ShapeDtypeStruct((7, 49, 49, D), jnp.float32),
    )(mt, dt, wt, ht, mnt)
    return c4.reshape(7 ** 5, D)


def kernel(x, minute_table, hour_table, weekday_table, day_table, month_table):
    b, t, _ = x.shape
    c = _build_combined(month_table[:7], day_table[:7], weekday_table[:7],
                        hour_table[:7], minute_table[:7])
    return _sc_lookup(x.astype(jnp.int32), c, b)


# speedup vs baseline: 2.0092x; 1.8734x over previous
import functools

import jax
import jax.numpy as jnp
from jax import lax
from jax.experimental import pallas as pl
from jax.experimental.pallas import tpu as pltpu
from jax.experimental.pallas import tpu_sc as plsc

D = 128
T = 200
CPA = 96
CPB = 104
NC = 2
NS = 16
NW = NC * NS
NBUF = 8
GD = 3


def _sc_lookup(keys2d, c_table, nb):
    rows_per_w = nb // NW
    iters = rows_per_w * 2
    groups = iters // NBUF
    assert nb % NW == 0 and iters % NBUF == 0 and groups >= 2
    mesh = plsc.VectorSubcoreMesh(core_axis_name="c", subcore_axis_name="s")

    def _cp(b):
        return CPA if b % 2 == 0 else CPB

    scratch = (
        [pltpu.VMEM((_cp(b),), jnp.int32) for b in range(NBUF)]
        + [pltpu.VMEM((_cp(b), D), jnp.float32) for b in range(NBUF)]
        + [pltpu.SemaphoreType.DMA for _ in range(3 * NBUF)]
    )

    @functools.partial(
        pl.kernel,
        out_type=jax.ShapeDtypeStruct((nb, T, D), jnp.float32),
        mesh=mesh,
        scratch_types=scratch,
        compiler_params=pltpu.CompilerParams(needs_layout_passes=False),
    )
    def k(k_hbm, c_hbm, out_hbm, *refs):
        keys = refs[0:NBUF]
        rows = refs[NBUF:2 * NBUF]
        sk = refs[2 * NBUF:3 * NBUF]
        sg = refs[3 * NBUF:4 * NBUF]
        sw = refs[4 * NBUF:5 * NBUF]
        wid = lax.axis_index("s") * NC + lax.axis_index("c")
        wrow = wid * rows_per_w

        def rh(g, b):
            return wrow + g // 2, 0 if b % 2 == 0 else CPA

        def fire_kread(g, b):
            r, h = rh(g, b)
            pltpu.async_copy(k_hbm.at[pl.ds(r * T + h, _cp(b))], keys[b], sk[b])

        def wait_kread(b):
            pltpu.make_async_copy(k_hbm.at[pl.ds(0, _cp(b))], keys[b],
                                  sk[b]).wait()

        def fire_gather(b):
            pltpu.async_copy(c_hbm.at[keys[b]], rows[b], sg[b])

        def wait_gather(b):
            pltpu.make_async_copy(c_hbm.at[keys[b]], rows[b], sg[b]).wait()

        def fire_write(g, b):
            r, h = rh(g, b)
            pltpu.async_copy(rows[b], out_hbm.at[r, pl.ds(h, _cp(b)), :], sw[b])

        def wait_write(b):
            pltpu.make_async_copy(rows[b], out_hbm.at[0, pl.ds(0, _cp(b)), :],
                                  sw[b]).wait()

        def step(g, b, fire_next, wait_w, drain):
            wait_kread(b)
            if wait_w:
                wait_write(b)
            fire_gather(b)
            if drain:
                pb = (b - GD) % NBUF
                wait_gather(pb)
                fire_write(g - GD, pb)
                if fire_next:
                    fire_kread(g - GD + NBUF, pb)

        for b in range(NBUF):
            fire_kread(b, b)
        for b in range(NBUF):
            step(b, b, fire_next=True, wait_w=False, drain=(b >= GD))

        def body(grp, c):
            g0 = grp * NBUF
            for b in range(NBUF):
                step(g0 + b, b, fire_next=True, wait_w=True, drain=True)
            return c

        lax.fori_loop(1, groups - 1, body, 0)

        gl = (groups - 1) * NBUF
        for b in range(NBUF):
            step(gl + b, b, fire_next=(gl + b < iters - NBUF + GD),
                 wait_w=True, drain=True)

        for i in range(GD):
            b = (NBUF - GD + i) % NBUF
            wait_gather(b)
            fire_write(iters - GD + i, b)
        for b in range(NBUF):
            wait_write(b)

    return k(keys2d, c_table)


def _build_combined(mt, dt, wt, ht, mnt):
    def body(m_ref, d_ref, w_ref, h_ref, mi_ref, out_ref):
        d_, w_, h_, mi_ = d_ref[...], w_ref[...], h_ref[...], mi_ref[...]
        m_row = m_ref[pl.ds(pl.program_id(0), 1), :]
        dw = (d_[:, None, :] + w_[None, :, :]).reshape(49, D)
        hm = (h_[:, None, :] + mi_[None, :, :]).reshape(49, D)
        out_ref[...] = ((m_row[0] + dw)[None, :, None, :]
                        + hm[None, None, :, :])

    row7 = pl.BlockSpec((7, D), lambda m: (0, 0))
    c4 = pl.pallas_call(
        body,
        grid=(7,),
        in_specs=[row7, row7, row7, row7, row7],
        out_specs=pl.BlockSpec((1, 49, 49, D), lambda m: (m, 0, 0, 0)),
        out_shape=jax.ShapeDtypeStruct((7, 49, 49, D), jnp.float32),
    )(mt, dt, wt, ht, mnt)
    return c4.reshape(7 ** 5, D)


def kernel(x, minute_table, hour_table, weekday_table, day_table, month_table):
    b, t, _ = x.shape
    c = _build_combined(month_table[:7], day_table[:7], weekday_table[:7],
                        hour_table[:7], minute_table[:7])
    xi = x.astype(jnp.int32)
    keys2d = ((((xi[:, :, 0] * 7 + xi[:, :, 1]) * 7 + xi[:, :, 2]) * 7
               + xi[:, :, 3]) * 7 + xi[:, :, 4])
    return _sc_lookup(keys2d.reshape(b * t), c, b)
